# baseline (speedup 1.0000x reference)
import functools

import jax
import jax.numpy as jnp
from jax import lax
from jax.experimental import pallas as pl
from jax.experimental.pallas import tpu as pltpu
from jax.experimental.pallas import tpu_sc as plsc

D_MODEL = 64
SCALE = 8.0
NUM_CORES = 2
NUM_SUBCORES = 16
NUM_WORKERS = NUM_CORES * NUM_SUBCORES
LANES = 16
BB = 128
NBUF = 4
LOOKAHEAD = 2

B = 4096
L = 200
N_UNITS = L * (B // BB)
UNITS_PER_W = N_UNITS // NUM_WORKERS
BT = B // BB


@jax.jit
def _emb_lookup(idx_flat, table):
    mesh = plsc.VectorSubcoreMesh(core_axis_name="c", subcore_axis_name="s")

    @functools.partial(
        pl.kernel,
        mesh=mesh,
        out_type=jax.ShapeDtypeStruct((L, D_MODEL // 8, BT, 8, BB), jnp.float32),
        scratch_types=[
            pltpu.VMEM((UNITS_PER_W * BB,), jnp.int32),
            [pltpu.VMEM((BB, D_MODEL), jnp.float32) for _ in range(NBUF)],
            [pltpu.VMEM((D_MODEL // 8, 8, BB), jnp.float32) for _ in range(NBUF)],
            [pltpu.SemaphoreType.DMA for _ in range(NBUF)],
            [pltpu.SemaphoreType.DMA for _ in range(NBUF)],
        ],
        compiler_params=pltpu.CompilerParams(
            use_tc_tiling_on_sc=False, needs_layout_passes=False
        ),
    )
    def k(idx_hbm, table_hbm, out_hbm, idx_v, gbufs, sbufs, sem_g, sem_w):
        wid = lax.axis_index("s") * NUM_CORES + lax.axis_index("c")
        base_tok = wid * UNITS_PER_W * BB
        pltpu.sync_copy(idx_hbm.at[pl.ds(base_tok, UNITS_PER_W * BB)], idx_v)
        iota = lax.iota(jnp.int32, LANES)

        def unit_lbt(g):
            u = wid * UNITS_PER_W + g
            return u // BT, u % BT

        def fire_gather(g, b):
            pltpu.async_copy(
                table_hbm.at[idx_v.at[pl.ds(g * BB, BB)]], gbufs[b], sem_g[b]
            )

        def wait_gather(g, b):
            pltpu.make_async_copy(
                table_hbm.at[idx_v.at[pl.ds(g * BB, BB)]], gbufs[b], sem_g[b]
            ).wait()

        def fire_writes(g, b):
            l, bt = unit_lbt(g)
            for dt in range(D_MODEL // 8):
                pltpu.async_copy(
                    sbufs[b].at[dt], out_hbm.at[l, dt, bt], sem_w[b]
                )

        def wait_writes(g, b):
            l, bt = unit_lbt(g)
            for dt in range(D_MODEL // 8):
                pltpu.make_async_copy(
                    sbufs[b].at[dt], out_hbm.at[l, dt, bt], sem_w[b]
                ).wait()

        for j in range(LOOKAHEAD):
            fire_gather(j, j)

        def round_body(r, carry):
            for j in range(NBUF):
                g = r * NBUF + j
                wait_gather(g, j)

                gb, sb = gbufs[j], sbufs[j]

                @plsc.parallel_loop(0, D_MODEL, 1, unroll=4)
                def _transp(d):
                    dt = d // 8
                    dr = d % 8
                    col = jnp.full((LANES,), d, jnp.int32)
                    for tg in range(BB // LANES):
                        row = iota + (tg * LANES)
                        v = plsc.load_gather(gb, [row, col])
                        sb[dt, dr, pl.ds(tg * LANES, LANES)] = v * SCALE

                fire_writes(g, j)

                nb = (j + LOOKAHEAD) % NBUF

                @pl.when(g + LOOKAHEAD < UNITS_PER_W)
                def _():
                    @pl.when(g >= NBUF - LOOKAHEAD)
                    def _():
                        wait_writes(g + LOOKAHEAD - NBUF, nb)

                    fire_gather(g + LOOKAHEAD, nb)

            return carry

        lax.fori_loop(0, UNITS_PER_W // NBUF, round_body, 0)

        for j in range(NBUF):
            wait_writes(UNITS_PER_W - NBUF + j, (UNITS_PER_W - NBUF + j) % NBUF)

    return k(idx_flat, table)


def kernel(x, table):
    b, l = x.shape
    idx_flat = x.T.reshape(b * l).astype(jnp.int32)
    raw = _emb_lookup(idx_flat, table)
    return raw.transpose(2, 4, 0, 1, 3).reshape(b, l, D_MODEL)


# speedup vs baseline: 1.0070x; 1.0070x over previous
import functools

import jax
import jax.numpy as jnp
from jax import lax
from jax.experimental import pallas as pl
from jax.experimental.pallas import tpu as pltpu
from jax.experimental.pallas import tpu_sc as plsc

D_MODEL = 64
SCALE = 8.0
NUM_CORES = 2
NUM_SUBCORES = 16
NUM_WORKERS = NUM_CORES * NUM_SUBCORES
LANES = 16
BB = 128
NBUF = 4
LOOKAHEAD = 2

B = 4096
L = 200
N_UNITS = L * (B // BB)
UNITS_PER_W = N_UNITS // NUM_WORKERS
BT = B // BB


@jax.jit
def _emb_lookup(idx_flat, table):
    mesh = plsc.VectorSubcoreMesh(core_axis_name="c", subcore_axis_name="s")

    @functools.partial(
        pl.kernel,
        mesh=mesh,
        out_type=jax.ShapeDtypeStruct((L, D_MODEL // 8, BT, 8, BB), jnp.float32),
        scratch_types=[
            pltpu.VMEM((UNITS_PER_W * BB,), jnp.int32),
            [pltpu.VMEM((BB, D_MODEL), jnp.float32) for _ in range(NBUF)],
            [pltpu.VMEM((D_MODEL // 8, 8, BB), jnp.float32) for _ in range(NBUF)],
            [pltpu.SemaphoreType.DMA for _ in range(NBUF)],
            [pltpu.SemaphoreType.DMA for _ in range(NBUF)],
        ],
        compiler_params=pltpu.CompilerParams(
            use_tc_tiling_on_sc=False, needs_layout_passes=False
        ),
    )
    def k(idx_hbm, table_hbm, out_hbm, idx_v, gbufs, sbufs, sem_g, sem_w):
        wid = lax.axis_index("s") * NUM_CORES + lax.axis_index("c")
        base_tok = wid * UNITS_PER_W * BB
        pltpu.sync_copy(idx_hbm.at[pl.ds(base_tok, UNITS_PER_W * BB)], idx_v)
        iota = lax.iota(jnp.int32, LANES)
        rows = [iota + (tg * LANES) for tg in range(BB // LANES)]

        def unit_lbt(g):
            u = wid * UNITS_PER_W + g
            return u // BT, u % BT

        def fire_gather(g, b):
            pltpu.async_copy(
                table_hbm.at[idx_v.at[pl.ds(g * BB, BB)]], gbufs[b], sem_g[b]
            )

        def wait_gather(g, b):
            pltpu.make_async_copy(
                table_hbm.at[idx_v.at[pl.ds(g * BB, BB)]], gbufs[b], sem_g[b]
            ).wait()

        def fire_writes(g, b):
            l, bt = unit_lbt(g)
            pltpu.async_copy(sbufs[b], out_hbm.at[l, :, bt], sem_w[b])

        def wait_writes(g, b):
            l, bt = unit_lbt(g)
            pltpu.make_async_copy(
                sbufs[b], out_hbm.at[l, :, bt], sem_w[b]
            ).wait()

        for j in range(LOOKAHEAD):
            fire_gather(j, j)

        def round_body(r, carry):
            for j in range(NBUF):
                g = r * NBUF + j
                wait_gather(g, j)

                gb, sb = gbufs[j], sbufs[j]

                @plsc.parallel_loop(0, D_MODEL, 1, unroll=4)
                def _transp(d):
                    dt = d // 8
                    dr = d % 8
                    col = jnp.full((LANES,), d, jnp.int32)
                    for tg in range(BB // LANES):
                        v = plsc.load_gather(gb, [rows[tg], col])
                        sb[dt, dr, pl.ds(tg * LANES, LANES)] = v * SCALE

                fire_writes(g, j)

                nb = (j + LOOKAHEAD) % NBUF

                @pl.when(g + LOOKAHEAD < UNITS_PER_W)
                def _():
                    @pl.when(g >= NBUF - LOOKAHEAD)
                    def _():
                        wait_writes(g + LOOKAHEAD - NBUF, nb)

                    fire_gather(g + LOOKAHEAD, nb)

            return carry

        lax.fori_loop(0, UNITS_PER_W // NBUF, round_body, 0)

        for j in range(NBUF):
            wait_writes(UNITS_PER_W - NBUF + j, (UNITS_PER_W - NBUF + j) % NBUF)

    return k(idx_flat, table)


def kernel(x, table):
    b, l = x.shape
    idx_flat = x.T.reshape(b * l).astype(jnp.int32)
    raw = _emb_lookup(idx_flat, table)
    return raw.transpose(2, 4, 0, 1, 3).reshape(b, l, D_MODEL)
